# baseline (speedup 1.0000x reference)
import functools

import jax
import jax.numpy as jnp
from jax import lax
from jax.experimental import pallas as pl
from jax.experimental.pallas import tpu as pltpu
from jax.experimental.pallas import tpu_sc as plsc

N_NODES = 10000
N_PAD = 10240
D = 128
E = 320000
CHUNK = 128
NCORE = 2
NSUB = 16
NW = NCORE * NSUB
CPT = 80
E_PAD = NW * CPT * CHUNK
NBLK = N_PAD // CHUNK
CPT0 = 40
CPT1 = 120
BPT = NBLK // NSUB

_MESH = plsc.VectorSubcoreMesh(
    core_axis_name="c", subcore_axis_name="s", num_cores=NCORE, num_subcores=NSUB
)


@functools.partial(
    pl.kernel,
    out_type=jax.ShapeDtypeStruct((NCORE, N_PAD, D), jnp.float32),
    mesh=_MESH,
    scratch_types=[
        pltpu.VMEM((CHUNK,), jnp.int32),
        pltpu.VMEM((CHUNK,), jnp.int32),
        pltpu.VMEM((CHUNK, D), jnp.float32),
        pltpu.VMEM((CHUNK, D), jnp.float32),
        pltpu.VMEM_SHARED((N_PAD, D), jnp.float32),
        pltpu.SemaphoreType.DMA,
    ],
)
def _deg_call(dst_hbm, rid_hbm, out_hbm, di_v, rid_v, ones_v, buf_v, acc_sh, sem):
    c = lax.axis_index("c")
    s = lax.axis_index("s")
    wid = s * NCORE + c

    z16 = jnp.zeros((16,), jnp.float32)
    e16 = jnp.where(lax.iota(jnp.int32, 16) == 0, 1.0, 0.0).astype(jnp.float32)

    def init(i, _):
        ones_v[i, pl.ds(0, 16)] = e16
        buf_v[i, pl.ds(0, 16)] = z16
        for k in range(1, D // 16):
            ones_v[i, pl.ds(k * 16, 16)] = z16
            buf_v[i, pl.ds(k * 16, 16)] = z16
        return 0
    lax.fori_loop(0, CHUNK, init, 0)

    def zero_blk(t, _):
        pltpu.sync_copy(rid_hbm.at[s * BPT + t], rid_v)
        pltpu.sync_copy(buf_v, acc_sh.at[rid_v])
        return 0
    lax.fori_loop(0, BPT, zero_blk, 0)
    plsc.subcore_barrier()

    def body(j, _):
        pltpu.sync_copy(dst_hbm.at[wid, j], di_v)
        pltpu.sync_copy(ones_v, acc_sh.at[di_v], add=True)
        return 0
    lax.fori_loop(0, CPT, body, 0)
    plsc.subcore_barrier()

    def copy_blk(t, _):
        blk = s * BPT + t
        pltpu.sync_copy(rid_hbm.at[blk], rid_v)
        pltpu.async_copy(acc_sh.at[rid_v], buf_v, sem).wait()
        pltpu.sync_copy(buf_v, out_hbm.at[c, pl.ds(blk * CHUNK, CHUNK)])
        return 0
    lax.fori_loop(0, BPT, copy_blk, 0)


@functools.partial(
    pl.kernel,
    out_type=jax.ShapeDtypeStruct((NCORE, N_PAD, D), jnp.float32),
    mesh=_MESH,
    scratch_types=[
        pltpu.VMEM((CHUNK,), jnp.int32),
        pltpu.VMEM((CHUNK,), jnp.int32),
        pltpu.VMEM((CHUNK,), jnp.int32),
        pltpu.VMEM((CHUNK,), jnp.int32),
        pltpu.VMEM((CHUNK,), jnp.int32),
        pltpu.VMEM((CHUNK, D), jnp.float32),
        pltpu.VMEM((CHUNK, D), jnp.float32),
        pltpu.VMEM_SHARED((N_PAD, D), jnp.float32),
        pltpu.SemaphoreType.DMA,
        pltpu.SemaphoreType.DMA,
        pltpu.SemaphoreType.DMA,
    ],
)
def _agg_call(h_hbm, src_hbm, dst_hbm, rid_hbm, out_hbm,
              si0, si1, di0, di1, rid_v, rows0, rows1, acc_sh,
              semg0, semg1, semi):
    c = lax.axis_index("c")
    s = lax.axis_index("s")

    cpt = jnp.where(c == 0, CPT0, CPT1)
    base = jnp.where(c == 0, s * CPT0, NSUB * CPT0 + s * CPT1)

    z16 = jnp.zeros((16,), jnp.float32)

    def init(i, _):
        for k in range(D // 16):
            rows0[i, pl.ds(k * 16, 16)] = z16
        return 0
    lax.fori_loop(0, CHUNK, init, 0)

    def zero_blk(t, _):
        pltpu.sync_copy(rid_hbm.at[s * BPT + t], rid_v)
        pltpu.sync_copy(rows0, acc_sh.at[rid_v])
        return 0
    lax.fori_loop(0, BPT, zero_blk, 0)
    plsc.subcore_barrier()

    pltpu.sync_copy(src_hbm.at[base], si0)
    pltpu.sync_copy(dst_hbm.at[base], di0)
    pltpu.async_copy(h_hbm.at[si0], rows0, semg0)

    def body(t, _):
        j1 = base + 2 * t + 1
        j2 = base + lax.rem(2 * t + 2, cpt)
        pltpu.async_copy(src_hbm.at[j1], si1, semi)
        pltpu.async_copy(dst_hbm.at[j1], di1, semi)
        pltpu.make_async_copy(h_hbm.at[si0], rows0, semg0).wait()
        pltpu.make_async_copy(src_hbm.at[j1], si1, semi).wait()
        pltpu.make_async_copy(dst_hbm.at[j1], di1, semi).wait()
        pltpu.async_copy(h_hbm.at[si1], rows1, semg1)
        pltpu.sync_copy(rows0, acc_sh.at[di0], add=True)
        pltpu.async_copy(src_hbm.at[j2], si0, semi)
        pltpu.async_copy(dst_hbm.at[j2], di0, semi)
        pltpu.make_async_copy(h_hbm.at[si1], rows1, semg1).wait()
        pltpu.make_async_copy(src_hbm.at[j2], si0, semi).wait()
        pltpu.make_async_copy(dst_hbm.at[j2], di0, semi).wait()
        pltpu.async_copy(h_hbm.at[si0], rows0, semg0)
        pltpu.sync_copy(rows1, acc_sh.at[di1], add=True)
        return 0
    lax.fori_loop(0, lax.div(cpt, 2), body, 0)

    pltpu.make_async_copy(h_hbm.at[si0], rows0, semg0).wait()
    plsc.subcore_barrier()

    def copy_blk(t, _):
        blk = s * BPT + t
        pltpu.sync_copy(rid_hbm.at[blk], rid_v)
        pltpu.async_copy(acc_sh.at[rid_v], rows0, semg0).wait()
        pltpu.sync_copy(rows0, out_hbm.at[c, pl.ds(blk * CHUNK, CHUNK)])
        return 0
    lax.fori_loop(0, BPT, copy_blk, 0)


_BM = 1280
_GRID = N_PAD // _BM


def _scale_mm_body(x_ref, w_ref, d0_ref, d1_ref, o_ref):
    d = lax.rsqrt(d0_ref[...] + d1_ref[...] + 1.0)
    o_ref[...] = jnp.dot(x_ref[...], w_ref[...], preferred_element_type=jnp.float32) * d


def _mid_body(a0_ref, a1_ref, hp_ref, d0_ref, d1_ref, w_ref, b_ref, o_ref):
    d = lax.rsqrt(d0_ref[...] + d1_ref[...] + 1.0)
    x2 = jnp.maximum(d * (a0_ref[0] + a1_ref[0] + hp_ref[...]) + b_ref[...], 0.0)
    h2 = jnp.dot(x2, w_ref[...], preferred_element_type=jnp.float32) * d
    i = pl.program_id(0)
    rows = i * _BM + lax.broadcasted_iota(jnp.int32, (_BM, 1), 0)
    o_ref[...] = jnp.where(rows < N_NODES, h2, 0.0)


def _final_body(a0_ref, a1_ref, hp_ref, d0_ref, d1_ref, b_ref, wc_ref, bc_ref, o_ref):
    d = lax.rsqrt(d0_ref[...] + d1_ref[...] + 1.0)
    x3 = jnp.maximum(d * (a0_ref[0] + a1_ref[0] + hp_ref[...]) + b_ref[...], 0.0)
    lg = jnp.dot(x3, wc_ref[...], preferred_element_type=jnp.float32) + bc_ref[...]
    m = jnp.max(lg, axis=1, keepdims=True)
    ssum = jnp.sum(jnp.exp(lg - m), axis=1, keepdims=True)
    o_ref[...] = lg - m - jnp.log(ssum)


def _row_spec(bm, bn):
    return pl.BlockSpec((bm, bn), lambda i: (i, 0))


def _full_spec(bm, bn):
    return pl.BlockSpec((bm, bn), lambda i: (0, 0))


def _agg_spec(part):
    return pl.BlockSpec((1, _BM, D), lambda i, p=part: (p, i, 0))


def kernel(x, edge_index, W1, b1, W2, b2, Wc, bc):
    ei = edge_index.astype(jnp.int32)
    npad = E_PAD - E
    src = jnp.concatenate([ei[0], jnp.full((npad,), N_NODES, jnp.int32)])
    dst = jnp.concatenate([ei[1], jnp.full((npad,), N_PAD - 1, jnp.int32)])
    src3 = src.reshape(NW, CPT, CHUNK)
    dst3 = dst.reshape(NW, CPT, CHUNK)
    srcc = src.reshape(E_PAD // CHUNK, CHUNK)
    dstc = dst.reshape(E_PAD // CHUNK, CHUNK)
    rowids = jnp.arange(N_PAD, dtype=jnp.int32).reshape(NBLK, CHUNK)
    blkids = jnp.arange(NBLK, dtype=jnp.int32).reshape(NSUB, BPT)
    r80 = jnp.arange(NBLK, dtype=jnp.int32)
    xp = jnp.pad(x, ((0, N_PAD - N_NODES), (0, 0)))

    degp = _deg_call(dst3, rowids)
    d0 = degp[0, :, 0:1]
    d1 = degp[1, :, 0:1]

    hp1 = pl.pallas_call(
        _scale_mm_body,
        grid=(_GRID,),
        in_specs=[_row_spec(_BM, D), _full_spec(D, D), _row_spec(_BM, 1), _row_spec(_BM, 1)],
        out_specs=_row_spec(_BM, D),
        out_shape=jax.ShapeDtypeStruct((N_PAD, D), jnp.float32),
    )(xp, W1, d0, d1)

    aggp1 = _agg_call(hp1, srcc, dstc, rowids)

    hp2 = pl.pallas_call(
        _mid_body,
        grid=(_GRID,),
        in_specs=[_agg_spec(0), _agg_spec(1), _row_spec(_BM, D), _row_spec(_BM, 1),
                  _row_spec(_BM, 1), _full_spec(D, D), _full_spec(1, D)],
        out_specs=_row_spec(_BM, D),
        out_shape=jax.ShapeDtypeStruct((N_PAD, D), jnp.float32),
    )(aggp1, aggp1, hp1, d0, d1, W2, b1.reshape(1, D))

    aggp2 = _agg_call(hp2, srcc, dstc, rowids)

    Wcp = jnp.pad(Wc, ((0, 0), (0, D - Wc.shape[1])))
    bcp = jnp.concatenate([bc, jnp.full((D - bc.shape[0],), -1e30, jnp.float32)])

    outp = pl.pallas_call(
        _final_body,
        grid=(_GRID,),
        in_specs=[_agg_spec(0), _agg_spec(1), _row_spec(_BM, D), _row_spec(_BM, 1),
                  _row_spec(_BM, 1), _full_spec(1, D), _full_spec(D, D), _full_spec(1, D)],
        out_specs=_row_spec(_BM, D),
        out_shape=jax.ShapeDtypeStruct((N_PAD, D), jnp.float32),
    )(aggp2, aggp2, hp2, d0, d1, b2.reshape(1, D), Wcp, bcp.reshape(1, D))

    return outp[:N_NODES, : Wc.shape[1]]


# speedup vs baseline: 1.0646x; 1.0646x over previous
import functools

import jax
import jax.numpy as jnp
from jax import lax
from jax.experimental import pallas as pl
from jax.experimental.pallas import tpu as pltpu
from jax.experimental.pallas import tpu_sc as plsc

N_NODES = 10000
N_PAD = 10240
D = 128
E = 320000
CHUNK = 128
NCORE = 2
NSUB = 16
NW = NCORE * NSUB
CPT = 80
E_PAD = NW * CPT * CHUNK
NBLK = N_PAD // CHUNK
G2 = 16
BPT = NBLK // NSUB

_MESH = plsc.VectorSubcoreMesh(
    core_axis_name="c", subcore_axis_name="s", num_cores=NCORE, num_subcores=NSUB
)


@functools.partial(
    pl.kernel,
    out_type=jax.ShapeDtypeStruct((NCORE, N_PAD, D), jnp.float32),
    mesh=_MESH,
    scratch_types=[
        pltpu.VMEM((CHUNK,), jnp.int32),
        pltpu.VMEM((CHUNK,), jnp.int32),
        pltpu.VMEM((CHUNK, D), jnp.float32),
        pltpu.VMEM((CHUNK, D), jnp.float32),
        pltpu.VMEM_SHARED((N_PAD, D), jnp.float32),
        pltpu.SemaphoreType.DMA,
    ],
)
def _deg_call(dst_hbm, rid_hbm, out_hbm, di_v, rid_v, ones_v, buf_v, acc_sh, sem):
    c = lax.axis_index("c")
    s = lax.axis_index("s")
    wid = s * NCORE + c

    z16 = jnp.zeros((16,), jnp.float32)
    e16 = jnp.where(lax.iota(jnp.int32, 16) == 0, 1.0, 0.0).astype(jnp.float32)

    def init(i, _):
        ones_v[i, pl.ds(0, 16)] = e16
        buf_v[i, pl.ds(0, 16)] = z16
        for k in range(1, D // 16):
            ones_v[i, pl.ds(k * 16, 16)] = z16
            buf_v[i, pl.ds(k * 16, 16)] = z16
        return 0
    lax.fori_loop(0, CHUNK, init, 0)

    def zero_blk(t, _):
        pltpu.sync_copy(rid_hbm.at[s * BPT + t], rid_v)
        pltpu.sync_copy(buf_v, acc_sh.at[rid_v])
        return 0
    lax.fori_loop(0, BPT, zero_blk, 0)
    plsc.subcore_barrier()

    def body(j, _):
        pltpu.sync_copy(dst_hbm.at[wid, j], di_v)
        pltpu.sync_copy(ones_v, acc_sh.at[di_v], add=True)
        return 0
    lax.fori_loop(0, CPT, body, 0)
    plsc.subcore_barrier()

    def copy_blk(t, _):
        blk = s * BPT + t
        pltpu.sync_copy(rid_hbm.at[blk], rid_v)
        pltpu.async_copy(acc_sh.at[rid_v], buf_v, sem).wait()
        pltpu.sync_copy(buf_v, out_hbm.at[c, pl.ds(blk * CHUNK, CHUNK)])
        return 0
    lax.fori_loop(0, BPT, copy_blk, 0)


@functools.partial(
    pl.kernel,
    out_type=jax.ShapeDtypeStruct((NCORE, N_PAD, D), jnp.float32),
    mesh=_MESH,
    scratch_types=[
        pltpu.VMEM((G2, CHUNK), jnp.int32),
        pltpu.VMEM((G2, CHUNK), jnp.int32),
        pltpu.VMEM((CHUNK,), jnp.int32),
        pltpu.VMEM((CHUNK, D), jnp.float32),
        pltpu.VMEM((CHUNK, D), jnp.float32),
        pltpu.VMEM_SHARED((N_PAD, D), jnp.float32),
        pltpu.SemaphoreType.DMA,
        pltpu.SemaphoreType.DMA,
    ],
)
def _agg_call(h_hbm, src_hbm, dst_hbm, rid_hbm, out_hbm,
              sig, dig, rid_v, rows0, rows1, acc_sh, semg0, semg1):
    c = lax.axis_index("c")
    s = lax.axis_index("s")
    base = (s * NCORE + c) * CPT

    z16 = jnp.zeros((16,), jnp.float32)

    def init(i, _):
        for k in range(D // 16):
            rows0[i, pl.ds(k * 16, 16)] = z16
        return 0
    lax.fori_loop(0, CHUNK, init, 0)

    def zero_blk(t, _):
        pltpu.sync_copy(rid_hbm.at[s * BPT + t], rid_v)
        pltpu.sync_copy(rows0, acc_sh.at[rid_v])
        return 0
    lax.fori_loop(0, BPT, zero_blk, 0)
    plsc.subcore_barrier()

    rbufs = (rows0, rows1)
    sems = (semg0, semg1)

    def group(g, _):
        gb = base + g * G2
        pltpu.sync_copy(src_hbm.at[pl.ds(gb, G2)], sig)
        pltpu.sync_copy(dst_hbm.at[pl.ds(gb, G2)], dig)
        pltpu.async_copy(h_hbm.at[sig.at[0]], rows0, semg0)
        for j in range(G2):
            b = j & 1
            pltpu.make_async_copy(h_hbm.at[sig.at[j]], rbufs[b], sems[b]).wait()
            if j + 1 < G2:
                pltpu.async_copy(h_hbm.at[sig.at[j + 1]], rbufs[1 - b], sems[1 - b])
            pltpu.sync_copy(rbufs[b], acc_sh.at[dig.at[j]], add=True)
        return 0
    lax.fori_loop(0, CPT // G2, group, 0)
    plsc.subcore_barrier()

    def copy_blk(t, _):
        blk = s * BPT + t
        pltpu.sync_copy(rid_hbm.at[blk], rid_v)
        pltpu.async_copy(acc_sh.at[rid_v], rows0, semg0).wait()
        pltpu.sync_copy(rows0, out_hbm.at[c, pl.ds(blk * CHUNK, CHUNK)])
        return 0
    lax.fori_loop(0, BPT, copy_blk, 0)


_BM = 1280
_GRID = N_PAD // _BM


def _scale_mm_body(x_ref, w_ref, d0_ref, d1_ref, o_ref):
    d = lax.rsqrt(d0_ref[...] + d1_ref[...] + 1.0)
    o_ref[...] = jnp.dot(x_ref[...], w_ref[...], preferred_element_type=jnp.float32) * d


def _mid_body(a0_ref, a1_ref, hp_ref, d0_ref, d1_ref, w_ref, b_ref, o_ref):
    d = lax.rsqrt(d0_ref[...] + d1_ref[...] + 1.0)
    x2 = jnp.maximum(d * (a0_ref[0] + a1_ref[0] + hp_ref[...]) + b_ref[...], 0.0)
    h2 = jnp.dot(x2, w_ref[...], preferred_element_type=jnp.float32) * d
    i = pl.program_id(0)
    rows = i * _BM + lax.broadcasted_iota(jnp.int32, (_BM, 1), 0)
    o_ref[...] = jnp.where(rows < N_NODES, h2, 0.0)


def _final_body(a0_ref, a1_ref, hp_ref, d0_ref, d1_ref, b_ref, wc_ref, bc_ref, o_ref):
    d = lax.rsqrt(d0_ref[...] + d1_ref[...] + 1.0)
    x3 = jnp.maximum(d * (a0_ref[0] + a1_ref[0] + hp_ref[...]) + b_ref[...], 0.0)
    lg = jnp.dot(x3, wc_ref[...], preferred_element_type=jnp.float32) + bc_ref[...]
    m = jnp.max(lg, axis=1, keepdims=True)
    ssum = jnp.sum(jnp.exp(lg - m), axis=1, keepdims=True)
    o_ref[...] = lg - m - jnp.log(ssum)


def _row_spec(bm, bn):
    return pl.BlockSpec((bm, bn), lambda i: (i, 0))


def _full_spec(bm, bn):
    return pl.BlockSpec((bm, bn), lambda i: (0, 0))


def _agg_spec(part):
    return pl.BlockSpec((1, _BM, D), lambda i, p=part: (p, i, 0))


def kernel(x, edge_index, W1, b1, W2, b2, Wc, bc):
    ei = edge_index.astype(jnp.int32)
    npad = E_PAD - E
    src = jnp.concatenate([ei[0], jnp.full((npad,), N_NODES, jnp.int32)])
    dst = jnp.concatenate([ei[1], jnp.full((npad,), N_PAD - 1, jnp.int32)])
    src3 = src.reshape(NW, CPT, CHUNK)
    dst3 = dst.reshape(NW, CPT, CHUNK)
    srcc = src.reshape(E_PAD // CHUNK, CHUNK)
    dstc = dst.reshape(E_PAD // CHUNK, CHUNK)
    rowids = jnp.arange(N_PAD, dtype=jnp.int32).reshape(NBLK, CHUNK)
    blkids = jnp.arange(NBLK, dtype=jnp.int32).reshape(NSUB, BPT)
    r80 = jnp.arange(NBLK, dtype=jnp.int32)
    xp = jnp.pad(x, ((0, N_PAD - N_NODES), (0, 0)))

    degp = _deg_call(dst3, rowids)
    d0 = degp[0, :, 0:1]
    d1 = degp[1, :, 0:1]

    hp1 = pl.pallas_call(
        _scale_mm_body,
        grid=(_GRID,),
        in_specs=[_row_spec(_BM, D), _full_spec(D, D), _row_spec(_BM, 1), _row_spec(_BM, 1)],
        out_specs=_row_spec(_BM, D),
        out_shape=jax.ShapeDtypeStruct((N_PAD, D), jnp.float32),
    )(xp, W1, d0, d1)

    aggp1 = _agg_call(hp1, srcc, dstc, rowids)

    hp2 = pl.pallas_call(
        _mid_body,
        grid=(_GRID,),
        in_specs=[_agg_spec(0), _agg_spec(1), _row_spec(_BM, D), _row_spec(_BM, 1),
                  _row_spec(_BM, 1), _full_spec(D, D), _full_spec(1, D)],
        out_specs=_row_spec(_BM, D),
        out_shape=jax.ShapeDtypeStruct((N_PAD, D), jnp.float32),
    )(aggp1, aggp1, hp1, d0, d1, W2, b1.reshape(1, D))

    aggp2 = _agg_call(hp2, srcc, dstc, rowids)

    Wcp = jnp.pad(Wc, ((0, 0), (0, D - Wc.shape[1])))
    bcp = jnp.concatenate([bc, jnp.full((D - bc.shape[0],), -1e30, jnp.float32)])

    outp = pl.pallas_call(
        _final_body,
        grid=(_GRID,),
        in_specs=[_agg_spec(0), _agg_spec(1), _row_spec(_BM, D), _row_spec(_BM, 1),
                  _row_spec(_BM, 1), _full_spec(1, D), _full_spec(D, D), _full_spec(1, D)],
        out_specs=_row_spec(_BM, D),
        out_shape=jax.ShapeDtypeStruct((N_PAD, D), jnp.float32),
    )(aggp2, aggp2, hp2, d0, d1, b2.reshape(1, D), Wcp, bcp.reshape(1, D))

    return outp[:N_NODES, : Wc.shape[1]]


# speedup vs baseline: 1.2102x; 1.1367x over previous
import functools

import jax
import jax.numpy as jnp
from jax import lax
from jax.experimental import pallas as pl
from jax.experimental.pallas import tpu as pltpu
from jax.experimental.pallas import tpu_sc as plsc

N_NODES = 10000
N_PAD = 10240
D = 128
E = 320000
CHUNK = 128
NCORE = 2
NSUB = 16
NW = NCORE * NSUB
CPT = 80
E_PAD = NW * CPT * CHUNK
NBLK = N_PAD // CHUNK
BPT = NBLK // NSUB

_MESH = plsc.VectorSubcoreMesh(
    core_axis_name="c", subcore_axis_name="s", num_cores=NCORE, num_subcores=NSUB
)


@functools.partial(
    pl.kernel,
    out_type=jax.ShapeDtypeStruct((NCORE, N_PAD, D), jnp.float32),
    mesh=_MESH,
    scratch_types=[
        pltpu.VMEM((CHUNK,), jnp.int32),
        pltpu.VMEM((CHUNK,), jnp.int32),
        pltpu.VMEM((CHUNK, D), jnp.float32),
        pltpu.VMEM((CHUNK, D), jnp.float32),
        pltpu.VMEM_SHARED((N_PAD, D), jnp.float32),
        pltpu.SemaphoreType.DMA,
    ],
)
def _deg_call(dst_hbm, rid_hbm, out_hbm, di_v, rid_v, ones_v, buf_v, acc_sh, sem):
    c = lax.axis_index("c")
    s = lax.axis_index("s")
    wid = s * NCORE + c

    z16 = jnp.zeros((16,), jnp.float32)
    e16 = jnp.where(lax.iota(jnp.int32, 16) == 0, 1.0, 0.0).astype(jnp.float32)

    def init(i, _):
        ones_v[i, pl.ds(0, 16)] = e16
        buf_v[i, pl.ds(0, 16)] = z16
        for k in range(1, D // 16):
            ones_v[i, pl.ds(k * 16, 16)] = z16
            buf_v[i, pl.ds(k * 16, 16)] = z16
        return 0
    lax.fori_loop(0, CHUNK, init, 0)

    def zero_blk(t, _):
        pltpu.sync_copy(rid_hbm.at[s * BPT + t], rid_v)
        pltpu.sync_copy(buf_v, acc_sh.at[rid_v])
        return 0
    lax.fori_loop(0, BPT, zero_blk, 0)
    plsc.subcore_barrier()

    def body(j, _):
        pltpu.sync_copy(dst_hbm.at[wid, j], di_v)
        pltpu.sync_copy(ones_v, acc_sh.at[di_v], add=True)
        return 0
    lax.fori_loop(0, CPT, body, 0)
    plsc.subcore_barrier()

    def copy_blk(t, _):
        blk = s * BPT + t
        pltpu.sync_copy(rid_hbm.at[blk], rid_v)
        pltpu.async_copy(acc_sh.at[rid_v], buf_v, sem).wait()
        pltpu.sync_copy(buf_v, out_hbm.at[c, pl.ds(blk * CHUNK, CHUNK)])
        return 0
    lax.fori_loop(0, BPT, copy_blk, 0)


@functools.partial(
    pl.kernel,
    out_type=jax.ShapeDtypeStruct((NCORE, N_PAD, D), jnp.float32),
    mesh=_MESH,
    scratch_types=[
        pltpu.VMEM((CHUNK,), jnp.int32),
        pltpu.VMEM((CHUNK,), jnp.int32),
        pltpu.VMEM((CHUNK,), jnp.int32),
        pltpu.VMEM((CHUNK,), jnp.int32),
        pltpu.VMEM((CHUNK,), jnp.int32),
        pltpu.VMEM((CHUNK, D), jnp.float32),
        pltpu.VMEM((CHUNK, D), jnp.float32),
        pltpu.VMEM_SHARED((N_PAD, D), jnp.float32),
        pltpu.SemaphoreType.DMA,
        pltpu.SemaphoreType.DMA,
        pltpu.SemaphoreType.DMA,
    ],
)
def _agg_call(h_hbm, src_hbm, dst_hbm, rid_hbm, out_hbm,
              si0, si1, di0, di1, rid_v, rows0, rows1, acc_sh,
              semg0, semg1, semi):
    c = lax.axis_index("c")
    s = lax.axis_index("s")
    wid = s * NCORE + c

    z16 = jnp.zeros((16,), jnp.float32)

    def init(i, _):
        for k in range(D // 16):
            rows0[i, pl.ds(k * 16, 16)] = z16
        return 0
    lax.fori_loop(0, CHUNK, init, 0)

    def zero_blk(t, _):
        pltpu.sync_copy(rid_hbm.at[s * BPT + t], rid_v)
        pltpu.sync_copy(rows0, acc_sh.at[rid_v])
        return 0
    lax.fori_loop(0, BPT, zero_blk, 0)
    plsc.subcore_barrier()

    pltpu.sync_copy(src_hbm.at[wid, 0], si0)
    pltpu.sync_copy(dst_hbm.at[wid, 0], di0)
    pltpu.async_copy(h_hbm.at[si0], rows0, semg0)

    def body(t, _):
        j1 = 2 * t + 1
        j2 = lax.rem(2 * t + 2, CPT)
        pltpu.async_copy(src_hbm.at[wid, j1], si1, semi)
        pltpu.async_copy(dst_hbm.at[wid, j1], di1, semi)
        pltpu.make_async_copy(h_hbm.at[si0], rows0, semg0).wait()
        pltpu.make_async_copy(src_hbm.at[wid, j1], si1, semi).wait()
        pltpu.make_async_copy(dst_hbm.at[wid, j1], di1, semi).wait()
        pltpu.async_copy(h_hbm.at[si1], rows1, semg1)
        pltpu.sync_copy(rows0, acc_sh.at[di0], add=True)
        pltpu.async_copy(src_hbm.at[wid, j2], si0, semi)
        pltpu.async_copy(dst_hbm.at[wid, j2], di0, semi)
        pltpu.make_async_copy(h_hbm.at[si1], rows1, semg1).wait()
        pltpu.make_async_copy(src_hbm.at[wid, j2], si0, semi).wait()
        pltpu.make_async_copy(dst_hbm.at[wid, j2], di0, semi).wait()
        pltpu.async_copy(h_hbm.at[si0], rows0, semg0)
        pltpu.sync_copy(rows1, acc_sh.at[di1], add=True)
        return 0
    lax.fori_loop(0, CPT // 2, body, 0)

    pltpu.make_async_copy(h_hbm.at[si0], rows0, semg0).wait()
    plsc.subcore_barrier()

    def copy_blk(t, _):
        blk = s * BPT + t
        pltpu.sync_copy(rid_hbm.at[blk], rid_v)
        pltpu.async_copy(acc_sh.at[rid_v], rows0, semg0).wait()
        pltpu.sync_copy(rows0, out_hbm.at[c, pl.ds(blk * CHUNK, CHUNK)])
        return 0
    lax.fori_loop(0, BPT, copy_blk, 0)


_BM = 1280
_GRID = N_PAD // _BM


def _scale_mm_body(x_ref, w_ref, d0_ref, d1_ref, o_ref):
    d = lax.rsqrt(d0_ref[...] + d1_ref[...] + 1.0)
    o_ref[...] = jnp.dot(x_ref[...], w_ref[...], preferred_element_type=jnp.float32) * d


def _mid_body(a0_ref, a1_ref, hp_ref, d0_ref, d1_ref, w_ref, b_ref, o_ref):
    d = lax.rsqrt(d0_ref[...] + d1_ref[...] + 1.0)
    x2 = jnp.maximum(d * (a0_ref[0] + a1_ref[0] + hp_ref[...]) + b_ref[...], 0.0)
    h2 = jnp.dot(x2, w_ref[...], preferred_element_type=jnp.float32) * d
    i = pl.program_id(0)
    rows = i * _BM + lax.broadcasted_iota(jnp.int32, (_BM, 1), 0)
    o_ref[...] = jnp.where(rows < N_NODES, h2, 0.0)


def _final_body(a0_ref, a1_ref, hp_ref, d0_ref, d1_ref, b_ref, wc_ref, bc_ref, o_ref):
    d = lax.rsqrt(d0_ref[...] + d1_ref[...] + 1.0)
    x3 = jnp.maximum(d * (a0_ref[0] + a1_ref[0] + hp_ref[...]) + b_ref[...], 0.0)
    lg = jnp.dot(x3, wc_ref[...], preferred_element_type=jnp.float32) + bc_ref[...]
    m = jnp.max(lg, axis=1, keepdims=True)
    ssum = jnp.sum(jnp.exp(lg - m), axis=1, keepdims=True)
    o_ref[...] = lg - m - jnp.log(ssum)


def _row_spec(bm, bn):
    return pl.BlockSpec((bm, bn), lambda i: (i, 0))


def _full_spec(bm, bn):
    return pl.BlockSpec((bm, bn), lambda i: (0, 0))


def _agg_spec(part):
    return pl.BlockSpec((1, _BM, D), lambda i, p=part: (p, i, 0))


def kernel(x, edge_index, W1, b1, W2, b2, Wc, bc):
    ei = edge_index.astype(jnp.int32)
    npad = E_PAD - E
    src = jnp.concatenate([ei[0], jnp.full((npad,), N_NODES, jnp.int32)])
    dst = jnp.concatenate([ei[1], jnp.full((npad,), N_PAD - 1, jnp.int32)])
    src3 = src.reshape(NW, CPT, CHUNK)
    dst3 = dst.reshape(NW, CPT, CHUNK)
    rowids = jnp.arange(N_PAD, dtype=jnp.int32).reshape(NBLK, CHUNK)
    blkids = jnp.arange(NBLK, dtype=jnp.int32).reshape(NSUB, BPT)
    r80 = jnp.arange(NBLK, dtype=jnp.int32)
    xp = jnp.pad(x, ((0, N_PAD - N_NODES), (0, 0)))

    degp = _deg_call(dst3, rowids)
    d0 = degp[0, :, 0:1]
    d1 = degp[1, :, 0:1]

    hp1 = pl.pallas_call(
        _scale_mm_body,
        grid=(_GRID,),
        in_specs=[_row_spec(_BM, D), _full_spec(D, D), _row_spec(_BM, 1), _row_spec(_BM, 1)],
        out_specs=_row_spec(_BM, D),
        out_shape=jax.ShapeDtypeStruct((N_PAD, D), jnp.float32),
    )(xp, W1, d0, d1)

    aggp1 = _agg_call(hp1, src3, dst3, rowids)

    hp2 = pl.pallas_call(
        _mid_body,
        grid=(_GRID,),
        in_specs=[_agg_spec(0), _agg_spec(1), _row_spec(_BM, D), _row_spec(_BM, 1),
                  _row_spec(_BM, 1), _full_spec(D, D), _full_spec(1, D)],
        out_specs=_row_spec(_BM, D),
        out_shape=jax.ShapeDtypeStruct((N_PAD, D), jnp.float32),
    )(aggp1, aggp1, hp1, d0, d1, W2, b1.reshape(1, D))

    aggp2 = _agg_call(hp2, src3, dst3, rowids)

    Wcp = jnp.pad(Wc, ((0, 0), (0, D - Wc.shape[1])))
    bcp = jnp.concatenate([bc, jnp.full((D - bc.shape[0],), -1e30, jnp.float32)])

    outp = pl.pallas_call(
        _final_body,
        grid=(_GRID,),
        in_specs=[_agg_spec(0), _agg_spec(1), _row_spec(_BM, D), _row_spec(_BM, 1),
                  _row_spec(_BM, 1), _full_spec(1, D), _full_spec(D, D), _full_spec(1, D)],
        out_specs=_row_spec(_BM, D),
        out_shape=jax.ShapeDtypeStruct((N_PAD, D), jnp.float32),
    )(aggp2, aggp2, hp2, d0, d1, b2.reshape(1, D), Wcp, bcp.reshape(1, D))

    return outp[:N_NODES, : Wc.shape[1]]


# speedup vs baseline: 1.3623x; 1.1257x over previous
import functools

import jax
import jax.numpy as jnp
from jax import lax
from jax.experimental import pallas as pl
from jax.experimental.pallas import tpu as pltpu
from jax.experimental.pallas import tpu_sc as plsc

N_NODES = 10000
N_PAD = 10240
D = 128
E = 320000
CHUNK = 128
NCORE = 2
NSUB = 16
NW = NCORE * NSUB
CPT = 80
E_PAD = NW * CPT * CHUNK
NBLK = N_PAD // CHUNK
BPT = NBLK // NSUB

_MESH = plsc.VectorSubcoreMesh(
    core_axis_name="c", subcore_axis_name="s", num_cores=NCORE, num_subcores=NSUB
)


@functools.partial(
    pl.kernel,
    out_type=jax.ShapeDtypeStruct((NW, N_PAD), jnp.float32),
    mesh=_MESH,
    compiler_params=pltpu.CompilerParams(needs_layout_passes=False),
    scratch_types=[
        pltpu.VMEM((CPT * CHUNK,), jnp.int32),
        pltpu.VMEM((N_PAD,), jnp.float32),
    ],
)
def _deg_call(dst_hbm, out_hbm, di_v, hist_v):
    c = lax.axis_index("c")
    s = lax.axis_index("s")
    wid = s * NCORE + c

    z16 = jnp.zeros((16,), jnp.float32)
    o16 = jnp.ones((16,), jnp.float32)

    def init(i, _):
        hist_v[pl.ds(i * 16, 16)] = z16
        return 0
    lax.fori_loop(0, N_PAD // 16, init, 0)

    pltpu.sync_copy(dst_hbm.at[wid], di_v)

    def hbody(i, _):
        idx = di_v[pl.ds(i * 16, 16)]
        plsc.addupdate_scatter(hist_v, [idx], o16)
        return 0
    lax.fori_loop(0, CPT * CHUNK // 16, hbody, 0)

    pltpu.sync_copy(hist_v, out_hbm.at[wid])


@functools.partial(
    pl.kernel,
    out_type=jax.ShapeDtypeStruct((NCORE, N_PAD, D), jnp.float32),
    mesh=_MESH,
    scratch_types=[
        pltpu.VMEM((CHUNK,), jnp.int32),
        pltpu.VMEM((CHUNK,), jnp.int32),
        pltpu.VMEM((CHUNK,), jnp.int32),
        pltpu.VMEM((CHUNK,), jnp.int32),
        pltpu.VMEM((CHUNK,), jnp.int32),
        pltpu.VMEM((CHUNK, D), jnp.float32),
        pltpu.VMEM((CHUNK, D), jnp.float32),
        pltpu.VMEM_SHARED((N_PAD, D), jnp.float32),
        pltpu.SemaphoreType.DMA,
        pltpu.SemaphoreType.DMA,
        pltpu.SemaphoreType.DMA,
    ],
)
def _agg_call(h_hbm, src_hbm, dst_hbm, rid_hbm, out_hbm,
              si0, si1, di0, di1, rid_v, rows0, rows1, acc_sh,
              semg0, semg1, semi):
    c = lax.axis_index("c")
    s = lax.axis_index("s")
    wid = s * NCORE + c

    z16 = jnp.zeros((16,), jnp.float32)

    def init(i, _):
        for k in range(D // 16):
            rows0[i, pl.ds(k * 16, 16)] = z16
        return 0
    lax.fori_loop(0, CHUNK, init, 0)

    def zero_blk(t, _):
        pltpu.sync_copy(rid_hbm.at[s * BPT + t], rid_v)
        pltpu.sync_copy(rows0, acc_sh.at[rid_v])
        return 0
    lax.fori_loop(0, BPT, zero_blk, 0)
    plsc.subcore_barrier()

    pltpu.sync_copy(src_hbm.at[wid, 0], si0)
    pltpu.sync_copy(dst_hbm.at[wid, 0], di0)
    pltpu.async_copy(h_hbm.at[si0], rows0, semg0)

    def body(t, _):
        j1 = 2 * t + 1
        j2 = lax.rem(2 * t + 2, CPT)
        pltpu.async_copy(src_hbm.at[wid, j1], si1, semi)
        pltpu.async_copy(dst_hbm.at[wid, j1], di1, semi)
        pltpu.make_async_copy(h_hbm.at[si0], rows0, semg0).wait()
        pltpu.make_async_copy(src_hbm.at[wid, j1], si1, semi).wait()
        pltpu.make_async_copy(dst_hbm.at[wid, j1], di1, semi).wait()
        pltpu.async_copy(h_hbm.at[si1], rows1, semg1)
        pltpu.sync_copy(rows0, acc_sh.at[di0], add=True)
        pltpu.async_copy(src_hbm.at[wid, j2], si0, semi)
        pltpu.async_copy(dst_hbm.at[wid, j2], di0, semi)
        pltpu.make_async_copy(h_hbm.at[si1], rows1, semg1).wait()
        pltpu.make_async_copy(src_hbm.at[wid, j2], si0, semi).wait()
        pltpu.make_async_copy(dst_hbm.at[wid, j2], di0, semi).wait()
        pltpu.async_copy(h_hbm.at[si0], rows0, semg0)
        pltpu.sync_copy(rows1, acc_sh.at[di1], add=True)
        return 0
    lax.fori_loop(0, CPT // 2, body, 0)

    pltpu.make_async_copy(h_hbm.at[si0], rows0, semg0).wait()
    plsc.subcore_barrier()

    def copy_blk(t, _):
        blk = s * BPT + t
        pltpu.sync_copy(rid_hbm.at[blk], rid_v)
        pltpu.async_copy(acc_sh.at[rid_v], rows0, semg0).wait()
        pltpu.sync_copy(rows0, out_hbm.at[c, pl.ds(blk * CHUNK, CHUNK)])
        return 0
    lax.fori_loop(0, BPT, copy_blk, 0)


_BM = 1280
_GRID = N_PAD // _BM


def _scale_mm_body(x_ref, w_ref, dt_ref, o_ref):
    d = lax.rsqrt(jnp.sum(dt_ref[...], axis=1, keepdims=True) + 1.0)
    o_ref[...] = jnp.dot(x_ref[...], w_ref[...], preferred_element_type=jnp.float32) * d


def _mid_body(a0_ref, a1_ref, hp_ref, dt_ref, w_ref, b_ref, o_ref):
    d = lax.rsqrt(jnp.sum(dt_ref[...], axis=1, keepdims=True) + 1.0)
    x2 = jnp.maximum(d * (a0_ref[0] + a1_ref[0] + hp_ref[...]) + b_ref[...], 0.0)
    h2 = jnp.dot(x2, w_ref[...], preferred_element_type=jnp.float32) * d
    i = pl.program_id(0)
    rows = i * _BM + lax.broadcasted_iota(jnp.int32, (_BM, 1), 0)
    o_ref[...] = jnp.where(rows < N_NODES, h2, 0.0)


def _final_body(a0_ref, a1_ref, hp_ref, dt_ref, b_ref, wc_ref, bc_ref, o_ref):
    d = lax.rsqrt(jnp.sum(dt_ref[...], axis=1, keepdims=True) + 1.0)
    x3 = jnp.maximum(d * (a0_ref[0] + a1_ref[0] + hp_ref[...]) + b_ref[...], 0.0)
    lg = jnp.dot(x3, wc_ref[...], preferred_element_type=jnp.float32) + bc_ref[...]
    m = jnp.max(lg, axis=1, keepdims=True)
    ssum = jnp.sum(jnp.exp(lg - m), axis=1, keepdims=True)
    o_ref[...] = lg - m - jnp.log(ssum)


def _row_spec(bm, bn):
    return pl.BlockSpec((bm, bn), lambda i: (i, 0))


def _full_spec(bm, bn):
    return pl.BlockSpec((bm, bn), lambda i: (0, 0))


def _agg_spec(part):
    return pl.BlockSpec((1, _BM, D), lambda i, p=part: (p, i, 0))


def kernel(x, edge_index, W1, b1, W2, b2, Wc, bc):
    ei = edge_index.astype(jnp.int32)
    npad = E_PAD - E
    src = jnp.concatenate([ei[0], jnp.full((npad,), N_NODES, jnp.int32)])
    dst = jnp.concatenate([ei[1], jnp.full((npad,), N_PAD - 1, jnp.int32)])
    src3 = src.reshape(NW, CPT, CHUNK)
    dst3 = dst.reshape(NW, CPT, CHUNK)
    rowids = jnp.arange(N_PAD, dtype=jnp.int32).reshape(NBLK, CHUNK)
    xp = jnp.pad(x, ((0, N_PAD - N_NODES), (0, 0)))

    dst2 = dst.reshape(NW, CPT * CHUNK)
    degp = _deg_call(dst2)
    degT = degp.T

    hp1 = pl.pallas_call(
        _scale_mm_body,
        grid=(_GRID,),
        in_specs=[_row_spec(_BM, D), _full_spec(D, D), _row_spec(_BM, NW)],
        out_specs=_row_spec(_BM, D),
        out_shape=jax.ShapeDtypeStruct((N_PAD, D), jnp.float32),
    )(xp, W1, degT)

    aggp1 = _agg_call(hp1, src3, dst3, rowids)

    hp2 = pl.pallas_call(
        _mid_body,
        grid=(_GRID,),
        in_specs=[_agg_spec(0), _agg_spec(1), _row_spec(_BM, D), _row_spec(_BM, NW),
                  _full_spec(D, D), _full_spec(1, D)],
        out_specs=_row_spec(_BM, D),
        out_shape=jax.ShapeDtypeStruct((N_PAD, D), jnp.float32),
    )(aggp1, aggp1, hp1, degT, W2, b1.reshape(1, D))

    aggp2 = _agg_call(hp2, src3, dst3, rowids)

    Wcp = jnp.pad(Wc, ((0, 0), (0, D - Wc.shape[1])))
    bcp = jnp.concatenate([bc, jnp.full((D - bc.shape[0],), -1e30, jnp.float32)])

    outp = pl.pallas_call(
        _final_body,
        grid=(_GRID,),
        in_specs=[_agg_spec(0), _agg_spec(1), _row_spec(_BM, D), _row_spec(_BM, NW),
                  _full_spec(1, D), _full_spec(D, D), _full_spec(1, D)],
        out_specs=_row_spec(_BM, D),
        out_shape=jax.ShapeDtypeStruct((N_PAD, D), jnp.float32),
    )(aggp2, aggp2, hp2, degT, b2.reshape(1, D), Wcp, bcp.reshape(1, D))

    return outp[:N_NODES, : Wc.shape[1]]
